# Initial kernel scaffold; baseline (speedup 1.0000x reference)
import jax
import jax.numpy as jnp
from jax.experimental import pallas as pl


def kernel(x, Wg, w1, b1, w2, b2, ws1, bs1, ws2, bs2):
    raise NotImplementedError("write your pallas kernel here")


# speedup vs baseline: 1.3557x; 1.3557x over previous
import jax
import jax.numpy as jnp
from jax.experimental import pallas as pl
from jax.experimental.pallas import tpu as pltpu

N = 2048
D = 1024
F = 2048
E = 8
TM = 512
T = N // TM


def _gate_shared_body(x_ref, wg_ref, ws1_ref, bs1_ref, ws2_ref, bs2_ref,
                      comb_ref, sh_ref):
    xt = x_ref[...]
    logits = jax.lax.dot_general(
        xt, wg_ref[...], (((1,), (1,)), ((), ())),
        preferred_element_type=jnp.float32)
    m = jnp.max(logits, axis=1, keepdims=True)
    ex = jnp.exp(logits - m)
    s = ex / jnp.sum(ex, axis=1, keepdims=True)
    lane8 = jax.lax.broadcasted_iota(jnp.int32, (TM, E), 1)
    m1 = jnp.max(s, axis=1, keepdims=True)
    i1 = jnp.min(jnp.where(s == m1, lane8, E), axis=1, keepdims=True)
    s2 = jnp.where(lane8 == i1, -1e30, s)
    m2 = jnp.max(s2, axis=1, keepdims=True)
    i2 = jnp.min(jnp.where(s2 == m2, lane8, E), axis=1, keepdims=True)
    denom = m1 + m2 + 1e-20
    w1v = m1 / denom
    w2v = m2 / denom
    lane16 = jax.lax.broadcasted_iota(jnp.int32, (TM, 16), 1)
    comb = (jnp.where(lane16 == i1, w1v, 0.0) +
            jnp.where(lane16 == i2, w2v, 0.0))
    comb_ref[...] = comb
    h1 = jnp.maximum(
        jnp.dot(xt, ws1_ref[...], preferred_element_type=jnp.float32)
        + bs1_ref[...], 0.0)
    sh_ref[...] = (jnp.dot(h1, ws2_ref[...], preferred_element_type=jnp.float32)
                   + bs2_ref[...])


def _experts_body(comb_ref, x_ref, sh_ref, w1_ref, b1_ref, w2_ref, b2_ref,
                  out_ref):
    e = pl.program_id(0)
    t = pl.program_id(1)
    rows = pl.ds(t * TM, TM)
    xt = x_ref[rows, :]
    lane16 = jax.lax.broadcasted_iota(jnp.int32, (TM, 16), 1)
    col = jnp.sum(jnp.where(lane16 == e, comb_ref[...], 0.0),
                  axis=1, keepdims=True)
    h1 = jnp.maximum(
        jnp.dot(xt, w1_ref[0], preferred_element_type=jnp.float32)
        + b1_ref[0], 0.0)
    contrib = (jnp.dot(h1 * col, w2_ref[0], preferred_element_type=jnp.float32)
               + col * b2_ref[0])

    @pl.when(e == 0)
    def _():
        out_ref[rows, :] = sh_ref[...] + contrib

    @pl.when(e > 0)
    def _():
        out_ref[rows, :] += contrib


def kernel(x, Wg, w1, b1, w2, b2, ws1, bs1, ws2, bs2):
    bsz, seq_len, h = x.shape
    xf = x.reshape(N, D)
    bs1r = bs1.reshape(1, F)
    bs2r = bs2.reshape(1, D)
    b1r = b1.reshape(E, 1, F)
    b2r = b2.reshape(E, 1, D)

    comb, sh = pl.pallas_call(
        _gate_shared_body,
        grid=(T,),
        in_specs=[
            pl.BlockSpec((TM, D), lambda t: (t, 0)),
            pl.BlockSpec((E, D), lambda t: (0, 0)),
            pl.BlockSpec((D, F), lambda t: (0, 0)),
            pl.BlockSpec((1, F), lambda t: (0, 0)),
            pl.BlockSpec((F, D), lambda t: (0, 0)),
            pl.BlockSpec((1, D), lambda t: (0, 0)),
        ],
        out_specs=[
            pl.BlockSpec((TM, 16), lambda t: (t, 0)),
            pl.BlockSpec((TM, D), lambda t: (t, 0)),
        ],
        out_shape=[
            jax.ShapeDtypeStruct((N, 16), jnp.float32),
            jax.ShapeDtypeStruct((N, D), jnp.float32),
        ],
        compiler_params=pltpu.CompilerParams(
            dimension_semantics=("arbitrary",)),
    )(xf, Wg, ws1, bs1r, ws2, bs2r)

    out = pl.pallas_call(
        _experts_body,
        grid=(E, T),
        in_specs=[
            pl.BlockSpec((TM, 16), lambda e, t: (t, 0)),
            pl.BlockSpec((N, D), lambda e, t: (0, 0)),
            pl.BlockSpec((TM, D), lambda e, t: (t, 0)),
            pl.BlockSpec((1, D, F), lambda e, t: (e, 0, 0)),
            pl.BlockSpec((1, 1, F), lambda e, t: (e, 0, 0)),
            pl.BlockSpec((1, F, D), lambda e, t: (e, 0, 0)),
            pl.BlockSpec((1, 1, D), lambda e, t: (e, 0, 0)),
        ],
        out_specs=pl.BlockSpec((N, D), lambda e, t: (0, 0)),
        out_shape=jax.ShapeDtypeStruct((N, D), jnp.float32),
        compiler_params=pltpu.CompilerParams(
            dimension_semantics=("arbitrary", "arbitrary")),
    )(comb, xf, sh, w1, b1r, w2, b2r)

    return out.reshape(bsz, seq_len, h)
